# baseline (speedup 1.0000x reference)
import functools

import jax
import jax.numpy as jnp
from jax import lax
from jax.experimental import pallas as pl
from jax.experimental.pallas import tpu as pltpu
from jax.experimental.pallas import tpu_sc as plsc

N = 10000
E = 320000
D = 128

NC = 2
NS = 16
DH = D // NC
EPT = E // NS
CH = 80
NCHUNK = EPT // CH
NPRE = 1
NCPAD = NCHUNK + NPRE
EPTP = NCPAD * CH
N2 = 10240
NPT = N2 // NS

_mesh = plsc.VectorSubcoreMesh(
    core_axis_name="c", subcore_axis_name="s", num_cores=NC, num_subcores=NS)

_gdn = lax.GatherDimensionNumbers(
    offset_dims=(), collapsed_slice_dims=(0,), start_index_map=(0,))


def _splat(vec, l):
  idx = jnp.full((16,), l, dtype=jnp.int32)
  return lax.gather(vec, idx[:, None], _gdn, (1,),
                    mode=lax.GatherScatterMode.PROMISE_IN_BOUNDS)


@functools.partial(
    pl.kernel,
    out_type=jax.ShapeDtypeStruct((NC, N2, DH), jnp.float32),
    mesh=_mesh,
    scratch_types=[
        pltpu.VMEM((NCPAD, CH), jnp.int32),
        pltpu.VMEM((NCPAD, CH), jnp.int32),
        pltpu.VMEM((EPTP,), jnp.float32),
        pltpu.VMEM((CH, D), jnp.float32),
        pltpu.VMEM((CH, D), jnp.float32),
        pltpu.VMEM_SHARED((N2, DH), jnp.float32),
        pltpu.SemaphoreType.DMA,
        pltpu.SemaphoreType.DMA,
    ],
    compiler_params=pltpu.CompilerParams(use_tc_tiling_on_sc=False),
)
def _sc_segsum(hs_hbm, hfull_hbm, src_hbm, dst_hbm, attn_hbm, zeros_hbm, part_hbm,
               src_v, dst_v, attn_v, rows0_v, rows1_v,
               acc_sh, gsem, ssem):
  c = lax.axis_index("c")
  s = lax.axis_index("s")
  bufs = (rows0_v, rows1_v)

  pltpu.sync_copy(src_hbm.at[s], src_v)
  pltpu.sync_copy(dst_hbm.at[s], dst_v)
  pltpu.sync_copy(attn_hbm.at[s], attn_v)

  h_half = hfull_hbm

  def gather(i, buf):
    pltpu.async_copy(h_half.at[src_v.at[i]], buf, gsem)

  def scatter(i, buf):
    pltpu.async_copy(buf, acc_sh.at[dst_v.at[i]], ssem, add=True)

  def wait_gather():
    pltpu.make_async_copy(h_half.at[src_v.at[0]], rows0_v, gsem).wait()

  def wait_scatter():
    pltpu.make_async_copy(rows0_v, acc_sh.at[dst_v.at[0]], ssem).wait()

  def scale(i, buf):
    for g in range(CH // 16):
      a16 = attn_v[pl.ds(i * CH + g * 16, 16)]
      for l in range(16):
        asp = _splat(a16, l)
        e = g * 16 + l
        for j in range(DH // 16):
          buf[e, pl.ds(j * 16, 16)] = buf[e, pl.ds(j * 16, 16)] * asp

  gather(0, bufs[0])

  def body2(t, carry):
    for k in range(2):
      i = 2 * t + k
      wait_gather()

      gather(i + NPRE, bufs[(k + NPRE) % 2])
    return carry

  lax.fori_loop(0, NCHUNK // 2, body2, 0)
  wait_gather()
  plsc.subcore_barrier()

  pltpu.sync_copy(acc_sh.at[pl.ds(s * NPT, NPT)],
                  part_hbm.at[c, pl.ds(s * NPT, NPT)])


_ROWS = 400
_NBLK = N // _ROWS
_CN = (((1,), (1,)), ((), ()))


def _bi_compute(h, p0_ref, p1_ref, w1_ref, b1_ref, w2_ref, b2_ref):
  hn = jnp.concatenate([p0_ref[0], p1_ref[0]], axis=1)
  t1 = lax.dot_general(h + hn, w1_ref[...], _CN,
                       preferred_element_type=jnp.float32) + b1_ref[...]
  t2 = lax.dot_general(h * hn, w2_ref[...], _CN,
                       preferred_element_type=jnp.float32) + b2_ref[...]
  return (jnp.where(t1 > 0, t1, 0.01 * t1)
          + jnp.where(t2 > 0, t2, 0.01 * t2))


def _bi_mid_body(h_ref, p0_ref, p1_ref, w1_ref, b1_ref, w2_ref, b2_ref,
                 os_ref):
  out = _bi_compute(h_ref[...], p0_ref, p1_ref, w1_ref, b1_ref, w2_ref,
                    b2_ref)
  os_ref[0] = out[:, :DH]
  os_ref[1] = out[:, DH:]


def _bi_last_body(x_ref, h1s0_ref, h1s1_ref, p0_ref, p1_ref,
                  w1_ref, b1_ref, w2_ref, b2_ref, o_ref):
  h1 = jnp.concatenate([h1s0_ref[0], h1s1_ref[0]], axis=1)
  out = _bi_compute(h1, p0_ref, p1_ref, w1_ref, b1_ref, w2_ref, b2_ref)
  o_ref[:, :D] = x_ref[...]
  o_ref[:, D:2 * D] = h1
  o_ref[:, 2 * D:] = out


def _half_spec(c):
  return pl.BlockSpec((1, _ROWS, DH), lambda i, c=c: (c, i, 0))


_WSPEC = pl.BlockSpec((D, D), lambda i: (0, 0))
_BSPEC = pl.BlockSpec((1, D), lambda i: (0, 0))


def _bi_mid(h, part, w1, b1, w2, b2):
  return pl.pallas_call(
      _bi_mid_body,
      grid=(_NBLK,),
      in_specs=[pl.BlockSpec((_ROWS, D), lambda i: (i, 0)),
                _half_spec(0), _half_spec(1), _WSPEC, _BSPEC, _WSPEC, _BSPEC],
      out_specs=pl.BlockSpec((NC, _ROWS, DH), lambda i: (0, i, 0)),
      out_shape=jax.ShapeDtypeStruct((NC, N, DH), jnp.float32),
  )(h, part, part, w1, b1.reshape(1, D), w2, b2.reshape(1, D))


def _bi_last(x, h1s, part, w1, b1, w2, b2):
  return pl.pallas_call(
      _bi_last_body,
      grid=(_NBLK,),
      in_specs=[pl.BlockSpec((_ROWS, D), lambda i: (i, 0)),
                _half_spec(0), _half_spec(1),
                _half_spec(0), _half_spec(1), _WSPEC, _BSPEC, _WSPEC, _BSPEC],
      out_specs=pl.BlockSpec((_ROWS, 3 * D), lambda i: (i, 0)),
      out_shape=jax.ShapeDtypeStruct((N, 3 * D), jnp.float32),
  )(x, h1s, h1s, part, part, w1, b1.reshape(1, D), w2, b2.reshape(1, D))


def kernel(x, edge_index, edge_attn,
           W1_0, b1_0, W2_0, b2_0, W1_1, b1_1, W2_1, b2_1):
  pad = EPTP - EPT
  src = jnp.pad(edge_index[0].reshape(NS, EPT), ((0, 0), (0, pad))
                ).reshape(NS, NCPAD, CH)
  trash = N + (jnp.arange(NS * pad, dtype=jnp.int32) % (N2 - N)).reshape(NS, pad)
  dst = jnp.concatenate(
      [edge_index[1].reshape(NS, EPT), trash], axis=1).reshape(NS, NCPAD, CH)
  attn = jnp.pad(edge_attn.reshape(NS, EPT), ((0, 0), (0, pad))
                 ).reshape(NS, EPTP)
  zeros = jnp.zeros((CH, DH), jnp.float32)

  xs = x.reshape(N, NC, DH).transpose(1, 0, 2)

  part = _sc_segsum(xs, x, src, dst, attn, zeros)
  h1s = _bi_mid(x, part, W1_0, b1_0, W2_0, b2_0)
  part = _sc_segsum(h1s, x, src, dst, attn, zeros)
  return _bi_last(x, h1s, part, W1_1, b1_1, W2_1, b2_1)


# speedup vs baseline: 1.5194x; 1.5194x over previous
import functools

import jax
import jax.numpy as jnp
from jax import lax
from jax.experimental import pallas as pl
from jax.experimental.pallas import tpu as pltpu
from jax.experimental.pallas import tpu_sc as plsc

N = 10000
E = 320000
D = 128

NC = 2
NS = 16
DH = D // NC
EPT = E // NS
CH = 80
NCHUNK = EPT // CH
NPRE = 1
NCPAD = NCHUNK + NPRE
EPTP = NCPAD * CH
N2 = 10240
NPT = N2 // NS

_mesh = plsc.VectorSubcoreMesh(
    core_axis_name="c", subcore_axis_name="s", num_cores=NC, num_subcores=NS)

_gdn = lax.GatherDimensionNumbers(
    offset_dims=(), collapsed_slice_dims=(0,), start_index_map=(0,))


def _splat(vec, l):
  idx = jnp.full((16,), l, dtype=jnp.int32)
  return lax.gather(vec, idx[:, None], _gdn, (1,),
                    mode=lax.GatherScatterMode.PROMISE_IN_BOUNDS)


@functools.partial(
    pl.kernel,
    out_type=jax.ShapeDtypeStruct((NC, N2, DH), jnp.float32),
    mesh=_mesh,
    scratch_types=[
        pltpu.VMEM((NCPAD, CH), jnp.int32),
        pltpu.VMEM((NCPAD, CH), jnp.int32),
        pltpu.VMEM((EPTP,), jnp.float32),
        pltpu.VMEM((CH, DH), jnp.bfloat16),
        pltpu.VMEM((CH, DH), jnp.bfloat16),
        pltpu.VMEM((CH, DH), jnp.float32),
        pltpu.VMEM((CH, DH), jnp.float32),
        pltpu.VMEM_SHARED((N2, DH), jnp.float32),
        pltpu.SemaphoreType.DMA,
        pltpu.SemaphoreType.DMA,
    ],
    compiler_params=pltpu.CompilerParams(use_tc_tiling_on_sc=False,
                                         needs_layout_passes=False),
)
def _sc_segsum(hs_hbm, src_hbm, dst_hbm, attn_hbm, zeros_hbm, part_hbm,
               src_v, dst_v, attn_v, rows0_v, rows1_v, srows0_v, srows1_v,
               acc_sh, gsem, ssem):
  c = lax.axis_index("c")
  s = lax.axis_index("s")
  bufs = (rows0_v, rows1_v)
  sbufs = (srows0_v, srows1_v)

  pltpu.sync_copy(zeros_hbm, srows1_v)
  for r in range(NPT // CH):
    pltpu.sync_copy(srows1_v, acc_sh.at[pl.ds(s * NPT + r * CH, CH)])
  plsc.subcore_barrier()

  pltpu.sync_copy(src_hbm.at[s], src_v)
  pltpu.sync_copy(dst_hbm.at[s], dst_v)
  pltpu.sync_copy(attn_hbm.at[s], attn_v)

  h_half = hs_hbm.at[c]

  def gather(i, buf):
    pltpu.async_copy(h_half.at[src_v.at[i]], buf, gsem)

  def scatter(i, sbuf):
    pltpu.async_copy(sbuf, acc_sh.at[dst_v.at[i]], ssem, add=True)

  def wait_gather():
    pltpu.make_async_copy(h_half.at[src_v.at[0]], rows0_v, gsem).wait()

  def wait_scatter():
    pltpu.make_async_copy(srows0_v, acc_sh.at[dst_v.at[0]], ssem).wait()

  _HI = jnp.full((16,), -65536, dtype=jnp.int32)

  def scale(i, buf, sbuf):
    for g in range(CH // 16):
      a16 = attn_v[pl.ds(i * CH + g * 16, 16)]
      for l in range(16):
        asp = _splat(a16, l)
        e = g * 16 + l
        for j in range(DH // 32):
          v = plsc.bitcast(buf[e, pl.ds(32 * j, 32)], jnp.int32)
          lo = plsc.bitcast(lax.shift_left(v, 16), jnp.float32)
          hi = plsc.bitcast(lax.bitwise_and(v, _HI), jnp.float32)
          sbuf[e, pl.ds(32 * j, 16)] = lo * asp
          sbuf[e, pl.ds(32 * j + 16, 16)] = hi * asp

  gather(0, bufs[0])

  def body2(t, carry):
    for k in range(2):
      i = 2 * t + k
      wait_gather()
      gather(i + NPRE, bufs[(k + NPRE) % 2])

      @pl.when(i > 0)
      def _():
        wait_scatter()

      scale(i, bufs[k], sbufs[k])
      scatter(i, sbufs[k])
    return carry

  lax.fori_loop(0, NCHUNK // 2, body2, 0)
  wait_gather()
  wait_scatter()
  plsc.subcore_barrier()

  pltpu.sync_copy(acc_sh.at[pl.ds(s * NPT, NPT)],
                  part_hbm.at[c, pl.ds(s * NPT, NPT)])


_ROWS = 400
_NBLK = N // _ROWS
_CN = (((1,), (1,)), ((), ()))


def _bi_compute(h, p0_ref, p1_ref, w1_ref, b1_ref, w2_ref, b2_ref):
  hn = jnp.concatenate([p0_ref[0], p1_ref[0]], axis=1)
  t1 = lax.dot_general(h + hn, w1_ref[...], _CN,
                       preferred_element_type=jnp.float32) + b1_ref[...]
  t2 = lax.dot_general(h * hn, w2_ref[...], _CN,
                       preferred_element_type=jnp.float32) + b2_ref[...]
  return (jnp.where(t1 > 0, t1, 0.01 * t1)
          + jnp.where(t2 > 0, t2, 0.01 * t2))


def _bi_mid_body(h_ref, p0_ref, p1_ref, w1_ref, b1_ref, w2_ref, b2_ref,
                 os_ref):
  out = _bi_compute(h_ref[...], p0_ref, p1_ref, w1_ref, b1_ref, w2_ref,
                    b2_ref)
  os_ref[0] = out[:, :DH]
  os_ref[1] = out[:, DH:]


def _bi_last_body(x_ref, h1s0_ref, h1s1_ref, p0_ref, p1_ref,
                  w1_ref, b1_ref, w2_ref, b2_ref, o_ref):
  h1 = jnp.concatenate([h1s0_ref[0], h1s1_ref[0]], axis=1)
  out = _bi_compute(h1, p0_ref, p1_ref, w1_ref, b1_ref, w2_ref, b2_ref)
  o_ref[:, :D] = x_ref[...]
  o_ref[:, D:2 * D] = h1
  o_ref[:, 2 * D:] = out


def _half_spec(c):
  return pl.BlockSpec((1, _ROWS, DH), lambda i, c=c: (c, i, 0))


_WSPEC = pl.BlockSpec((D, D), lambda i: (0, 0))
_BSPEC = pl.BlockSpec((1, D), lambda i: (0, 0))


def _bi_mid(h, part, w1, b1, w2, b2):
  return pl.pallas_call(
      _bi_mid_body,
      grid=(_NBLK,),
      in_specs=[pl.BlockSpec((_ROWS, D), lambda i: (i, 0)),
                _half_spec(0), _half_spec(1), _WSPEC, _BSPEC, _WSPEC, _BSPEC],
      out_specs=pl.BlockSpec((NC, _ROWS, DH), lambda i: (0, i, 0)),
      out_shape=jax.ShapeDtypeStruct((NC, N, DH), jnp.float32),
  )(h, part, part, w1, b1.reshape(1, D), w2, b2.reshape(1, D))


def _bi_last(x, h1s, part, w1, b1, w2, b2):
  return pl.pallas_call(
      _bi_last_body,
      grid=(_NBLK,),
      in_specs=[pl.BlockSpec((_ROWS, D), lambda i: (i, 0)),
                _half_spec(0), _half_spec(1),
                _half_spec(0), _half_spec(1), _WSPEC, _BSPEC, _WSPEC, _BSPEC],
      out_specs=pl.BlockSpec((_ROWS, 3 * D), lambda i: (i, 0)),
      out_shape=jax.ShapeDtypeStruct((N, 3 * D), jnp.float32),
  )(x, h1s, h1s, part, part, w1, b1.reshape(1, D), w2, b2.reshape(1, D))


import numpy as _np
_PI = _np.empty((DH,), dtype=_np.int32)
for _j in range(DH // 32):
  for _k in range(16):
    _PI[32 * _j + 2 * _k] = 32 * _j + _k
    _PI[32 * _j + 2 * _k + 1] = 32 * _j + 16 + _k
_PI = jnp.asarray(_PI)


def kernel(x, edge_index, edge_attn,
           W1_0, b1_0, W2_0, b2_0, W1_1, b1_1, W2_1, b2_1):
  pad = EPTP - EPT
  src = jnp.pad(edge_index[0].reshape(NS, EPT), ((0, 0), (0, pad))
                ).reshape(NS, NCPAD, CH)
  trash = N + (jnp.arange(NS * pad, dtype=jnp.int32) % (N2 - N)).reshape(NS, pad)
  dst = jnp.concatenate(
      [edge_index[1].reshape(NS, EPT), trash], axis=1).reshape(NS, NCPAD, CH)
  attn = jnp.pad(edge_attn.reshape(NS, EPT), ((0, 0), (0, pad))
                 ).reshape(NS, EPTP)
  zeros = jnp.zeros((CH, DH), jnp.float32)

  def to_sc(hs):
    return hs[:, :, _PI].astype(jnp.bfloat16)

  xs = x.reshape(N, NC, DH).transpose(1, 0, 2)

  part = _sc_segsum(to_sc(xs), src, dst, attn, zeros)
  h1s = _bi_mid(x, part, W1_0, b1_0, W2_0, b2_0)
  part = _sc_segsum(to_sc(h1s), src, dst, attn, zeros)
  return _bi_last(x, h1s, part, W1_1, b1_1, W2_1, b2_1)
